# baseline (speedup 1.0000x reference)
import functools

import jax
import jax.numpy as jnp
from jax import lax
from jax.experimental import pallas as pl
from jax.experimental.pallas import tpu as pltpu
from jax.experimental.pallas import tpu_sc as plsc

N = 10000
E = 320000
K = 3

NC = 2
NS = 16
LANES = 16

GRP = 128
IB = 1024
ACC_ROWS = N + 16
ZROWS = 312
WRITE_ROWS = 624

E_PAD = 327680
E_ALLOC = E_PAD + IB
EPC_E = E_PAD // (NC * NS)
NIB_E = EPC_E // IB
EPC_F = E_PAD // NS
NIB_F = EPC_F // IB
IDX_ROWS = E_ALLOC // GRP
NGB = 2


def _mesh():
  return plsc.VectorSubcoreMesh(core_axis_name="c", subcore_axis_name="s")


def _zero_acc(zero_hbm, acc, s):
  base = s * WRITE_ROWS
  pltpu.sync_copy(zero_hbm, acc.at[pl.ds(base, ZROWS), :])
  pltpu.sync_copy(zero_hbm, acc.at[pl.ds(base + ZROWS, ZROWS), :])

  @pl.when(s == NS - 1)
  def _():
    pltpu.sync_copy(zero_hbm.at[pl.ds(0, 32), :],
                    acc.at[pl.ds(NS * WRITE_ROWS, 32), :])


def _write_out(acc, out_ref, s):
  n0 = s * WRITE_ROWS
  pltpu.sync_copy(acc.at[pl.ds(n0, WRITE_ROWS), :],
                  out_ref.at[pl.ds(n0, WRITE_ROWS), :])

  @pl.when(s == NS - 1)
  def _():
    tail = NS * WRITE_ROWS
    pltpu.sync_copy(acc.at[pl.ds(tail, N - tail), :],
                    out_ref.at[pl.ds(tail, N - tail), :])


def _prop_pipelined(mode, u, srcs, dst2d):
  fh = 128
  gather = mode != "deg"
  fsplit = mode == "fsplit"
  nib = NIB_F if fsplit else NIB_E
  zeros_hbm = jnp.zeros((ZROWS, fh), jnp.float32)
  out_shape = (jax.ShapeDtypeStruct((2 * N, fh), jnp.float32) if fsplit
               else jax.ShapeDtypeStruct((NC, N, fh), jnp.float32))

  scratch = []
  scratch += [pltpu.VMEM((IB,), jnp.int32)] * (2 if gather else 0)
  scratch += [pltpu.VMEM((IB // GRP, GRP), jnp.int32)] * 2
  scratch += [pltpu.VMEM((GRP, fh), jnp.float32)] * (NGB if gather else 1)
  scratch += [pltpu.VMEM_SHARED((ACC_ROWS, fh), jnp.float32)]
  nsem = (2 if gather else 0) + 2 + (NGB if gather else 0) + NGB
  scratch += [pltpu.SemaphoreType.DMA] * nsem

  def body(*refs):
    if gather:
      u_hbm, src_hbm, dst_hbm, zero_hbm, out_hbm = refs[:5]
      rest = list(refs[5:])
      sidxb = [rest.pop(0), rest.pop(0)]
    else:
      dst_hbm, zero_hbm, one_hbm, out_hbm = refs[:4]
      rest = list(refs[4:])
      sidxb = None
    didxb = [rest.pop(0), rest.pop(0)]
    gbufs = [rest.pop(0) for _ in range(NGB if gather else 1)]
    acc = rest.pop(0)
    if gather:
      ssem = [rest.pop(0), rest.pop(0)]
      gsem = [rest.pop(0) for _ in range(NGB)]
    dsem = [rest.pop(0), rest.pop(0)]
    csem = [rest.pop(0) for _ in range(NGB)]

    c = lax.axis_index("c")
    s = lax.axis_index("s")
    _zero_acc(zero_hbm, acc, s)
    if not gather:
      pltpu.sync_copy(one_hbm, gbufs[0])
    plsc.subcore_barrier()

    if fsplit:
      w = s
      row_base = c * N
    else:
      w = c * NS + s
    e_base = w * (nib * IB)
    r_base = e_base // GRP

    def idx_copies(ib, par):
      r0 = pl.multiple_of(r_base + ib * (IB // GRP), 8)
      ds = pltpu.async_copy(
          dst_hbm.at[pl.ds(r0, IB // GRP), :], didxb[par], dsem[par])
      if gather:
        pltpu.async_copy(src_hbm.at[pl.ds(e_base + ib * IB, IB)],
                         sidxb[par], ssem[par])
      return ds

    idx_copies(0, 0)

    def pair_body(i, _):
      for par in (0, 1):
        ib = 2 * i + par
        pltpu.make_async_copy(
            dst_hbm.at[pl.ds(pl.multiple_of(r_base, 8), IB // GRP), :],
            didxb[par], dsem[par]).wait()
        if gather:
          pltpu.make_async_copy(
              src_hbm.at[pl.ds(e_base, IB)], sidxb[par], ssem[par]).wait()
        idx_copies(ib + 1, 1 - par)

        ngrp = IB // GRP
        gd = [None] * ngrp
        sd = [None] * ngrp
        if gather:
          for j in range(ngrp):
            b = j % NGB
            if j >= NGB:
              sd[j - NGB].wait()
            if fsplit:
              for k in range(GRP // LANES):
                sl = pl.ds(j * GRP + k * LANES, LANES)
                sidxb[par][sl] = sidxb[par][sl] + row_base
            gd[j] = pltpu.async_copy(
                u_hbm.at[sidxb[par].at[pl.ds(j * GRP, GRP)]],
                gbufs[b], gsem[b])
            if j >= 1:
              gd[j - 1].wait()
              sd[j - 1] = pltpu.async_copy(
                  gbufs[(j - 1) % NGB], acc.at[didxb[par].at[j - 1]],
                  csem[(j - 1) % NGB], add=True)
          gd[ngrp - 1].wait()
          sd[ngrp - 1] = pltpu.async_copy(
              gbufs[(ngrp - 1) % NGB], acc.at[didxb[par].at[ngrp - 1]],
              csem[(ngrp - 1) % NGB], add=True)
          for j in range(ngrp - NGB, ngrp):
            sd[j].wait()
        else:
          for j in range(ngrp):
            sd[j] = pltpu.async_copy(
                gbufs[0], acc.at[didxb[par].at[j]], csem[0], add=True)
          for j in range(ngrp):
            sd[j].wait()
      return 0

    lax.fori_loop(0, nib // 2, pair_body, 0)
    pltpu.make_async_copy(
        dst_hbm.at[pl.ds(pl.multiple_of(r_base, 8), IB // GRP), :],
        didxb[0], dsem[0]).wait()
    if gather:
      pltpu.make_async_copy(
          src_hbm.at[pl.ds(e_base, IB)], sidxb[0], ssem[0]).wait()
    plsc.subcore_barrier()
    if fsplit:
      _write_out(acc, out_hbm.at[pl.ds(row_base, N), :], s)
    else:
      _write_out(acc, out_hbm.at[c], s)

  kern = functools.partial(
      pl.kernel, out_type=out_shape, mesh=_mesh(), scratch_types=scratch)(body)
  if gather:
    return kern(u, srcs, dst2d, zeros_hbm)
  ones_hbm = jnp.ones((GRP, fh), jnp.float32)
  return kern(dst2d, zeros_hbm, ones_hbm)


def _matmul(a, w, b, relu):
  m, kd = a.shape
  fout = w.shape[1]
  bm = 1024

  def body(a_ref, w_ref, b_ref, o_ref):
    acc = jnp.dot(a_ref[...], w_ref[...],
                  preferred_element_type=jnp.float32) + b_ref[...]
    if relu:
      acc = jnp.maximum(acc, 0.0)
    o_ref[...] = acc

  return pl.pallas_call(
      body,
      grid=(pl.cdiv(m, bm),),
      in_specs=[
          pl.BlockSpec((bm, kd), lambda i: (i, 0)),
          pl.BlockSpec((kd, fout), lambda i: (0, 0)),
          pl.BlockSpec((1, fout), lambda i: (0, 0)),
      ],
      out_specs=pl.BlockSpec((bm, fout), lambda i: (i, 0)),
      out_shape=jax.ShapeDtypeStruct((m, fout), jnp.float32),
  )(a, w, b.reshape(1, fout))


def _to_split(t):
  n, f = t.shape
  return t.reshape(n, 2, f // 2).transpose(1, 0, 2).reshape(2 * n, f // 2)


def _from_split(t2):
  n2, fh = t2.shape
  n = n2 // 2
  return t2.reshape(2, n, fh).transpose(1, 0, 2).reshape(n, 2 * fh)


def kernel(x, edge_index, W1, b1, W2, b2, Wmu, bmu, Wls, bls):
  src = edge_index[0].astype(jnp.int32)
  dst = edge_index[1].astype(jnp.int32)
  pad = E_ALLOC - E
  srcs = jnp.concatenate([src, jnp.zeros((pad,), jnp.int32)])
  dst2d = jnp.concatenate(
      [dst, jnp.full((pad,), N, jnp.int32)]).reshape(IDX_ROWS, GRP)

  deg = _prop_pipelined("deg", None, srcs, dst2d).sum(axis=0)[:, 0]
  dinv = jnp.where(deg > 0, lax.rsqrt(jnp.maximum(deg, 1e-12)), 0.0)
  dcol = dinv[:, None]
  dsc = dinv[None, :, None]

  def S128(t):
    p = _prop_pipelined("esplit", dcol * t, srcs, dst2d).sum(axis=0)
    return dcol * p

  def S256(t2):
    u = (t2.reshape(2, N, 128) * dsc).reshape(2 * N, 128)
    p = _prop_pipelined("fsplit", u, srcs, dst2d)
    return (p.reshape(2, N, 128) * dsc).reshape(2 * N, 128)

  h1 = S128(x)
  h2 = S128(h1)
  h3 = S128(h2)
  a1 = jnp.concatenate([x, h1, h2, h3], axis=1)
  w1cat = W1.reshape((K + 1) * 128, 512)
  g1 = _matmul(a1, w1cat, b1, relu=True)

  w2cat = jnp.concatenate([W2[k] for k in range(K + 1)], axis=1)
  y = _matmul(g1, w2cat, jnp.zeros((1024,), jnp.float32), relu=False)
  y0, y1, y2_, y3 = (y[:, 256 * k:256 * (k + 1)] for k in range(4))
  t = S256(_to_split(y3))
  t = S256(_to_split(y2_) + t)
  t = S256(_to_split(y1) + t)
  g2 = jax.nn.relu(y0 + _from_split(t) + b2)

  whcat = jnp.concatenate(
      [jnp.concatenate([Wmu[k], Wls[k]], axis=1) for k in range(K + 1)],
      axis=1)
  z = _matmul(g2, whcat, jnp.zeros((512,), jnp.float32), relu=False)
  z0, z1, z2, z3 = (z[:, 128 * k:128 * (k + 1)] for k in range(4))
  t = S128(z3)
  t = S128(z2 + t)
  t = S128(z1 + t)
  out = z0 + t
  mu = out[:, :64] + bmu
  logstd = out[:, 64:] + bls

  std = jnp.exp(logstd)
  eps = jax.random.normal(jax.random.key(42), std.shape, dtype=std.dtype)
  zlat = eps * std + mu
  return (mu, logstd, zlat)
